# baseline (speedup 1.0000x reference)
import functools

import jax
import jax.numpy as jnp
from jax import lax
from jax.experimental import pallas as pl
from jax.experimental.pallas import tpu as pltpu
from jax.experimental.pallas import tpu_sc as plsc

_NC = 2
_NS = 16
_NW = _NC * _NS

_LANES = 16
_CH = 128


def _sc_pool(tpair, toff, rpair, roff, trig2, rate2, B, S, T, E, L):
    assert B % _NW == 0
    spw = B // _NW
    assert spw % 16 == 0
    nq = 4
    tpq = T // nq
    rows_q = tpq * S
    rpq = ((rows_q + _CH - 1) // _CH) * _CH
    nch = rpq // _CH
    ej = E // _LANES
    E2 = 2 * E
    rlp = ((L + _CH - 1) // _CH) * _CH
    assert _CH < L <= 2 * _CH

    mesh = plsc.VectorSubcoreMesh(core_axis_name="c", subcore_axis_name="s")

    @functools.partial(
        pl.kernel,
        out_type=(
            jax.ShapeDtypeStruct((B * T // 2, E2), jnp.float32),
            jax.ShapeDtypeStruct((B // 2, E2), jnp.float32),
        ),
        mesh=mesh,
        compiler_params=pltpu.CompilerParams(use_tc_tiling_on_sc=True),
        scratch_types=[
            pltpu.VMEM((rpq,), jnp.int32),
            pltpu.VMEM((rpq,), jnp.int32),
            pltpu.VMEM((rpq,), jnp.int32),
            pltpu.VMEM((rpq,), jnp.int32),
            pltpu.VMEM((rlp,), jnp.int32),
            pltpu.VMEM((rlp,), jnp.int32),
            pltpu.VMEM((rpq, E2), jnp.float32),
            pltpu.VMEM((rpq, E2), jnp.float32),
            pltpu.VMEM((_CH, E2), jnp.float32),
            pltpu.VMEM((T // 2, E2), jnp.float32),
            pltpu.VMEM((8, E2), jnp.float32),
            pltpu.SemaphoreType.DMA,
            pltpu.SemaphoreType.DMA,
            pltpu.SemaphoreType.DMA,
        ],
    )
    def pool(tp_hbm, to_hbm, rp_hbm, ro_hbm, tt_hbm, rt_hbm, tout_hbm,
             rout_hbm, idx_v0, idx_v1, off_v0, off_v1, ridx_v, roff_v,
             buf0, buf1, rbuf, featv, ratev, gsem0, gsem1, rsem):
        wid = lax.axis_index("s") * _NC + lax.axis_index("c")
        base_b = wid * spw
        idx_vs = (idx_v0, idx_v1)
        off_vs = (off_v0, off_v1)
        bufs = (buf0, buf1)
        gsems = (gsem0, gsem1)

        def fire_quarter(i, q, hb):
            start = ((base_b + i) * nq + q) * rpq
            pltpu.sync_copy(tp_hbm.at[pl.ds(start, rpq)], idx_vs[hb])
            pltpu.sync_copy(to_hbm.at[pl.ds(start, rpq)], off_vs[hb])
            for k in range(nch):
                pltpu.async_copy(
                    tt_hbm.at[idx_vs[hb].at[pl.ds(k * _CH, _CH)]],
                    bufs[hb].at[pl.ds(k * _CH, _CH)], gsems[hb])

        def wait_quarter(hb):
            pltpu.make_async_copy(
                tt_hbm.at[pl.ds(0, rpq)], bufs[hb], gsems[hb]).wait()

        def fire_rate_chunk(k):
            pltpu.async_copy(
                rt_hbm.at[ridx_v.at[pl.ds(k * _CH, _CH)]], rbuf, rsem)

        def wait_rate():
            pltpu.make_async_copy(
                rt_hbm.at[pl.ds(0, _CH)], rbuf, rsem).wait()

        def accum_quarter(q, hb):
            buf = bufs[hb]
            off_v = off_vs[hb]

            def tbody(tp, c):
                for u in range(2):
                    tt = tp * 2 + u
                    accs = [jnp.zeros((_LANES,), jnp.float32)
                            for _ in range(ej)]
                    for s in range(S):
                        r = tt * S + s
                        po = off_v[pl.ds(r, 16)][0]
                        for j in range(ej):
                            accs[j] = accs[j] + buf[r,
                                                    pl.ds(po + j * _LANES,
                                                          _LANES)]
                    for j in range(ej):
                        featv[q * (tpq // 2) + tp,
                              pl.ds(u * E + j * _LANES, _LANES)] = (
                            accs[j] * (1.0 / S))
                return c

            lax.fori_loop(0, tpq // 2, tbody, 0)

        def accum_rate_chunk(k, nrows, accs):
            base = k * _CH

            def rbody(s, a):
                po = roff_v[pl.ds(base + s, 16)][0]
                return tuple(
                    a[j] + rbuf[s, pl.ds(po + j * _LANES, _LANES)]
                    for j in range(ej))

            return lax.fori_loop(0, nrows, rbody, accs)

        fire_quarter(0, 0, 0)

        def sample_body(i, carry):
            b = base_b + i
            rstart = (base_b + i) * rlp
            pltpu.sync_copy(rp_hbm.at[pl.ds(rstart, rlp)], ridx_v)
            pltpu.sync_copy(ro_hbm.at[pl.ds(rstart, rlp)], roff_v)
            fire_rate_chunk(0)
            fire_quarter(i, 1, 1)
            wait_quarter(0)
            accum_quarter(0, 0)
            fire_quarter(i, 2, 0)
            wait_quarter(1)
            accum_quarter(1, 1)
            fire_quarter(i, 3, 1)
            wait_quarter(0)
            accum_quarter(2, 0)
            nxt = jnp.minimum(i + 1, spw - 1)
            fire_quarter(nxt, 0, 0)
            wait_quarter(1)
            accum_quarter(3, 1)
            trow = pl.multiple_of(b * (T // 2), 8)
            pltpu.sync_copy(featv, tout_hbm.at[pl.ds(trow, T // 2)])
            zero = tuple(jnp.zeros((_LANES,), jnp.float32)
                         for _ in range(ej))
            wait_rate()
            accs = accum_rate_chunk(0, _CH, zero)
            fire_rate_chunk(1)
            wait_rate()
            accs = accum_rate_chunk(1, L - _CH, accs)
            half = (i % 2) * E
            row = (i // 2) % 8
            for j in range(ej):
                ratev[row, pl.ds(half + j * _LANES, _LANES)] = (
                    accs[j] * (1.0 / L))
            rrow = pl.multiple_of(base_b // 2 + (i // 16) * 8, 8)
            pltpu.sync_copy(ratev, rout_hbm.at[pl.ds(rrow, 8)])
            return carry

        lax.fori_loop(0, spw, sample_body, 0)
        wait_quarter(0)

    return pool(tpair, toff, rpair, roff, trig2, rate2)


def _mlp(trig_feat, rate_feat, W1, b1, W2, b2, W3, b3, B, T, E, H, C):
    blk = 256
    assert B % blk == 0

    def body(tf_ref, rf_ref, w1_ref, b1_ref, w2a_ref, w2b_ref, b2_ref,
             w3_ref, b3_ref, o_ref):
        x = tf_ref[...]
        h1 = jnp.dot(x, w1_ref[...], preferred_element_type=jnp.float32)
        h1 = jnp.maximum(h1 + b1_ref[...], 0.0)
        h2 = (jnp.dot(rf_ref[...], w2a_ref[...],
                      preferred_element_type=jnp.float32)
              + jnp.dot(h1, w2b_ref[...], preferred_element_type=jnp.float32))
        h2 = jnp.maximum(h2 + b2_ref[...], 0.0)
        o_ref[...] = (jnp.dot(h2, w3_ref[...],
                              preferred_element_type=jnp.float32)
                      + b3_ref[...])

    grid = (B // blk,)
    full = lambda shape: pl.BlockSpec(shape, lambda i: (0,) * len(shape))
    return pl.pallas_call(
        body,
        grid=grid,
        in_specs=[
            pl.BlockSpec((blk, T * E), lambda i: (i, 0)),
            pl.BlockSpec((blk, E), lambda i: (i, 0)),
            full((T * E, T)),
            full((1, T)),
            full((E, H)),
            full((T, H)),
            full((1, H)),
            full((H, C)),
            full((1, C)),
        ],
        out_specs=pl.BlockSpec((blk, C), lambda i: (i, 0)),
        out_shape=jax.ShapeDtypeStruct((B, C), jnp.float32),
    )(trig_feat, rate_feat, W1, b1.reshape(1, T), W2[:E], W2[E:],
      b2.reshape(1, H), W3, b3.reshape(1, C))


def kernel(usr_trigram, usr_interacted_rates, trigram_table, rate_table,
           W1, b1, W2, b2, W3, b3):
    B, S, T = usr_trigram.shape
    L = usr_interacted_rates.shape[1]
    V, E = trigram_table.shape
    H = b2.shape[0]
    C = b3.shape[0]
    nq = 4
    rows_q = (T // nq) * S
    rpq = ((rows_q + _CH - 1) // _CH) * _CH
    rlp = ((L + _CH - 1) // _CH) * _CH

    trig2 = trigram_table.reshape(V // 2, 2 * E)
    rate2 = rate_table.reshape(V // 2, 2 * E)
    tq = usr_trigram.transpose(0, 2, 1).reshape(B, nq, rows_q)
    tq = jnp.pad(tq, ((0, 0), (0, 0), (0, rpq - rows_q)))
    tpair = (tq >> 1).reshape(B * nq * rpq)
    toff = ((tq & 1) * E).reshape(B * nq * rpq)
    rq = jnp.pad(usr_interacted_rates, ((0, 0), (0, rlp - L)))
    rpair = (rq >> 1).reshape(B * rlp)
    roff = ((rq & 1) * E).reshape(B * rlp)

    trig_feat, rate_feat = _sc_pool(
        tpair, toff, rpair, roff, trig2, rate2, B, S, T, E, L)
    trig_feat = trig_feat.reshape(B, T * E)
    rate_feat = rate_feat.reshape(B, E)
    return _mlp(trig_feat, rate_feat, W1, b1, W2, b2, W3, b3, B, T, E, H, C)


# speedup vs baseline: 6.9080x; 6.9080x over previous
import functools

import jax
import jax.numpy as jnp
from jax import lax
from jax.experimental import pallas as pl
from jax.experimental.pallas import tpu as pltpu
from jax.experimental.pallas import tpu_sc as plsc

_NC = 2
_NS = 16
_NW = _NC * _NS
_LANES = 16


def _sc_pool(trig_idx_t, rate_idx, trig_bf, rate_bf, B, S, T, E, L):
    assert B % (2 * _NW) == 0
    spw = B // _NW
    tph = T // 2
    rph = tph * S
    ch = 80
    assert rph % ch == 0 and ch % 8 == 0
    nch = rph // ch
    ej = E // _LANES
    rchunks = []
    off = 0
    while off < L:
        n = min(128, L - off)
        if L - off > 128:
            n -= n % 8
        rchunks.append((off, n))
        off += n

    mesh = plsc.VectorSubcoreMesh(core_axis_name="c", subcore_axis_name="s")

    @functools.partial(
        pl.kernel,
        out_type=(
            jax.ShapeDtypeStruct((B * T, E), jnp.float32),
            jax.ShapeDtypeStruct((B, E), jnp.float32),
        ),
        mesh=mesh,
        compiler_params=pltpu.CompilerParams(use_tc_tiling_on_sc=False,
                                             needs_layout_passes=False),
        scratch_types=[
            pltpu.VMEM((2, rph), jnp.int32),
            pltpu.VMEM((2, L), jnp.int32),
            pltpu.VMEM((2, rph, E), jnp.bfloat16),
            pltpu.VMEM((2, L, E), jnp.bfloat16),
            pltpu.VMEM((T, E), jnp.float32),
            pltpu.VMEM((1, E), jnp.float32),
            pltpu.SemaphoreType.DMA,
            pltpu.SemaphoreType.DMA,
            pltpu.SemaphoreType.DMA,
            pltpu.SemaphoreType.DMA,
        ],
    )
    def pool(ti_hbm, ri_hbm, tt_hbm, rt_hbm, tout_hbm, rout_hbm,
             idx_v, ridx_v, buf, rbuf, featv, ratev, gsem0, gsem1, rsem0,
             rsem1):
        wid = lax.axis_index("s") * _NC + lax.axis_index("c")
        base_b = wid * spw
        gsems = (gsem0, gsem1)
        rsems = (rsem0, rsem1)
        rps = T * S

        def fire_half(i, half, hb):
            start = (base_b + i) * rps + half * rph
            pltpu.sync_copy(ti_hbm.at[pl.ds(start, rph)], idx_v.at[hb])
            for k in range(nch):
                pltpu.async_copy(
                    tt_hbm.at[idx_v.at[hb, pl.ds(k * ch, ch)]],
                    buf.at[hb, pl.ds(k * ch, ch)], gsems[hb])

        def wait_half(hb):
            pltpu.make_async_copy(
                tt_hbm.at[pl.ds(0, rph)], buf.at[hb], gsems[hb]).wait()

        def fire_rate(i, rb):
            start = (base_b + i) * L
            pltpu.sync_copy(ri_hbm.at[pl.ds(start, L)], ridx_v.at[rb])
            for (o, n) in rchunks:
                pltpu.async_copy(
                    rt_hbm.at[ridx_v.at[rb, pl.ds(o, n)]],
                    rbuf.at[rb, pl.ds(o, n)], rsems[rb])

        def wait_rate(rb):
            pltpu.make_async_copy(
                rt_hbm.at[pl.ds(0, L)], rbuf.at[rb], rsems[rb]).wait()

        def row_terms(ref, *ix):
            terms = []
            for g in range(E // 32):
                packed = ref[(*ix, pl.ds(g * 32, 32))]
                a, b = plsc.unpack(packed,
                                   format=plsc.PackFormat.INTERLEAVED)
                terms.extend((a, b))
            return terms

        def accum_half(half, hb):
            def tbody(tt, c):
                accs = [jnp.zeros((_LANES,), jnp.float32) for _ in range(ej)]
                for s in range(S):
                    terms = row_terms(buf, hb, tt * S + s)
                    for j in range(ej):
                        accs[j] = accs[j] + terms[j]
                for j in range(ej):
                    featv[half * tph + tt, pl.ds(j * _LANES, _LANES)] = (
                        accs[j] * (1.0 / S))
                return c

            lax.fori_loop(0, tph, tbody, 0)

        def accum_rate(rb):
            def rbody(s, accs):
                terms = row_terms(rbuf, rb, s)
                return tuple(accs[j] + terms[j] for j in range(ej))

            raccs = lax.fori_loop(
                0, L, rbody,
                tuple(jnp.zeros((_LANES,), jnp.float32) for _ in range(ej)))
            for j in range(ej):
                ratev[0, pl.ds(j * _LANES, _LANES)] = raccs[j] * (1.0 / L)

        fire_half(0, 0, 0)
        fire_rate(0, 0)

        def pair_body(g, carry):
            for p in range(2):
                i = g * 2 + p
                b = base_b + i
                fire_half(i, 1, 1)
                nxt = jnp.minimum(i + 1, spw - 1)
                fire_rate(nxt, 1 - p)
                wait_half(0)
                accum_half(0, 0)
                fire_half(nxt, 0, 0)
                wait_half(1)
                accum_half(1, 1)
                pltpu.sync_copy(featv, tout_hbm.at[pl.ds(b * T, T)])
                wait_rate(p)
                accum_rate(p)
                pltpu.sync_copy(ratev, rout_hbm.at[pl.ds(b, 1)])
            return carry

        lax.fori_loop(0, spw // 2, pair_body, 0)
        wait_half(0)
        wait_rate(0)

    return pool(trig_idx_t, rate_idx, trig_bf, rate_bf)


def _mlp(trig_feat, rate_feat, W1p, b1, W2a, W2b, b2, W3, b3, B, T, E, H, C):
    blk = 256
    assert B % blk == 0

    def body(tf_ref, rf_ref, w1_ref, b1_ref, w2a_ref, w2b_ref, b2_ref,
             w3_ref, b3_ref, o_ref):
        x = tf_ref[...]
        h1 = jnp.dot(x, w1_ref[...], preferred_element_type=jnp.float32)
        h1 = jnp.maximum(h1 + b1_ref[...], 0.0)
        h2 = (jnp.dot(rf_ref[...], w2a_ref[...],
                      preferred_element_type=jnp.float32)
              + jnp.dot(h1, w2b_ref[...], preferred_element_type=jnp.float32))
        h2 = jnp.maximum(h2 + b2_ref[...], 0.0)
        o_ref[...] = (jnp.dot(h2, w3_ref[...],
                              preferred_element_type=jnp.float32)
                      + b3_ref[...])

    grid = (B // blk,)
    full = lambda shape: pl.BlockSpec(shape, lambda i: (0,) * len(shape))
    return pl.pallas_call(
        body,
        grid=grid,
        in_specs=[
            pl.BlockSpec((blk, T * E), lambda i: (i, 0)),
            pl.BlockSpec((blk, E), lambda i: (i, 0)),
            full((T * E, T)),
            full((1, T)),
            full((E, H)),
            full((T, H)),
            full((1, H)),
            full((H, C)),
            full((1, C)),
        ],
        out_specs=pl.BlockSpec((blk, C), lambda i: (i, 0)),
        out_shape=jax.ShapeDtypeStruct((B, C), jnp.float32),
    )(trig_feat, rate_feat, W1p, b1.reshape(1, T), W2a, W2b,
      b2.reshape(1, H), W3, b3.reshape(1, C))


def kernel(usr_trigram, usr_interacted_rates, trigram_table, rate_table,
           W1, b1, W2, b2, W3, b3):
    B, S, T = usr_trigram.shape
    L = usr_interacted_rates.shape[1]
    E = trigram_table.shape[1]
    H = b2.shape[0]
    C = b3.shape[0]

    trig_bf = trigram_table.astype(jnp.bfloat16)
    rate_bf = rate_table.astype(jnp.bfloat16)
    trig_idx_t = usr_trigram.transpose(0, 2, 1).reshape(B * T * S)
    rate_idx = usr_interacted_rates.reshape(B * L)

    trig_feat, rate_feat = _sc_pool(
        trig_idx_t, rate_idx, trig_bf, rate_bf, B, S, T, E, L)
    trig_feat = trig_feat.reshape(B, T * E)

    W1p = (W1.reshape(T, E // 32, 16, 2, T)
           .transpose(0, 1, 3, 2, 4).reshape(T * E, T))
    W2a = (W2[:E].reshape(E // 32, 16, 2, H)
           .transpose(0, 2, 1, 3).reshape(E, H))
    return _mlp(trig_feat, rate_feat, W1p, b1, W2a, W2[E:], b2, W3, b3,
                B, T, E, H, C)
